# Initial kernel scaffold; baseline (speedup 1.0000x reference)
import jax
import jax.numpy as jnp
from jax.experimental import pallas as pl


def kernel(feat, edge_index, W1, al1, ar1, b1, resW1, W2, al2, ar2, b2, W3, al3, ar3, b3, W4, al4, ar4, b4):
    raise NotImplementedError("write your pallas kernel here")


# speedup vs baseline: 3.3632x; 3.3632x over previous
import functools

import jax
import jax.numpy as jnp
from jax import lax
from jax.experimental import pallas as pl
from jax.experimental.pallas import tpu as pltpu
from jax.experimental.pallas import tpu_sc as plsc

N = 10000
E = 160000
D = 512

NW = 32
NSR = 80
R = 128
RPW = R // NW
NPAD = R * NSR
DUMMY = NSR

KB = 48
CE = KB * 43
PADE = E + CE + 16

_I16 = lambda: lax.iota(jnp.int32, 16)


def _dg(v, idx):
    return jnp.take_along_axis(v, idx, axis=0, mode="promise_in_bounds")


def _splat(v, j):
    return _dg(v, jnp.full((16,), j, dtype=jnp.int32))


def _sget_i32(ref, j):
    base = (j // 16) * 16
    v = ref[pl.ds(base, 16)]
    rot = _dg(v, (_I16() + (j - base)) & 15)
    return rot[0]


def _shift_up(v, d, fill):
    it = _I16()
    s = _dg(v, jnp.maximum(it - d, 0))
    return jnp.where(it >= d, s, fill)


def _seg_scan(key, val, op):
    it = _I16()
    for d in (1, 2, 4, 8):
        idx = jnp.maximum(it - d, 0)
        sk = _dg(key, idx)
        sv = _dg(val, idx)
        cond = (sk == key) & (it >= d)
        val = jnp.where(cond, op(val, sv), val)
    return val


def _sc_aggregate(ft, el, er_p, dst_p, src_p, ebnd):
    mesh = plsc.VectorSubcoreMesh(core_axis_name="c", subcore_axis_name="s",
                                  num_cores=2, num_subcores=16)

    @functools.partial(
        pl.kernel,
        out_type=jax.ShapeDtypeStruct((NPAD * D,), jnp.float32),
        mesh=mesh,
        scratch_types=[
            pltpu.VMEM((N,), jnp.float32),
            pltpu.VMEM((96,), jnp.float32),
            pltpu.VMEM((96,), jnp.float32),
            pltpu.VMEM((144,), jnp.int32),
            pltpu.VMEM((CE,), jnp.int32),
            pltpu.VMEM((CE,), jnp.int32),
            pltpu.VMEM((KB,), jnp.int32),
            pltpu.VMEM((KB, D), jnp.float32),
            pltpu.VMEM(((NSR + 1) * D,), jnp.float32),
            pltpu.SemaphoreType.DMA,
        ],
        compiler_params=pltpu.CompilerParams(needs_layout_passes=False,
                                             use_tc_tiling_on_sc=False),
    )
    def body(ft_h, el_h, er_h, dst_h, src_h, eb_h, out_h,
             el_v, er_v, em_v, eb_v, dst_v, src_v, idx_v, row_v, stg_v, sem):
        wid = lax.axis_index("s") * 2 + lax.axis_index("c")
        pltpu.sync_copy(el_h, el_v)
        pltpu.sync_copy(eb_h, eb_v)
        it = _I16()

        def range_body(r_i, _):
            r_glob = wid * RPW + r_i
            lo = r_glob * NSR
            es = _sget_i32(eb_v, r_glob)
            ee = _sget_i32(eb_v, r_glob + 1)
            a0 = (es // 8) * 8
            nedge = ee - es
            nck = (es - a0 + nedge + KB - 1) // KB
            nst = (nck + (CE // KB) - 1) // (CE // KB)

            pltpu.sync_copy(er_h.at[pl.ds(lo, 96)], er_v)

            def init_em(i, _c):
                em_v[pl.ds(i * 16, 16)] = jnp.full((16,), -3.4e38, jnp.float32)
                return _c
            lax.fori_loop(0, 6, init_em, 0)

            def init_stg(i, _c):
                stg_v[pl.ds(i * 16, 16)] = jnp.zeros((16,), jnp.float32)
                return _c
            lax.fori_loop(0, (NSR + 1) * D // 16, init_stg, 0)

            def load_keys(off):
                kraw = dst_v[pl.ds(off, 16)]
                sv = src_v[pl.ds(off, 16)]
                return kraw, sv

            def edge_logits(sv, kl):
                elg = plsc.load_gather(el_v, [sv])
                erg = plsc.load_gather(er_v, [kl])
                e = elg + erg
                return jnp.where(e > 0, e, 0.2 * e)

            def s1_stage(s, _c):
                pltpu.sync_copy(dst_h.at[pl.ds(a0 + s * CE, CE)], dst_v)
                pltpu.sync_copy(src_h.at[pl.ds(a0 + s * CE, CE)], src_v)

                def s1_vec(v, _c2):
                    off = v * 16
                    ge = a0 + s * CE + off + it
                    valid = (ge >= es) & (ge < ee)
                    kraw, sv = load_keys(off)
                    kl = jnp.where(valid, kraw - lo, DUMMY)
                    e = edge_logits(sv, kl)
                    mx = _seg_scan(kl, e, jnp.maximum)
                    knext = _dg(kl, jnp.minimum(it + 1, 15))
                    endm = (kl != knext) | (it == 15)
                    cur = plsc.load_gather(em_v, [kl])
                    plsc.store_scatter(em_v, [kl], jnp.maximum(mx, cur),
                                       mask=endm)
                    return _c2

                nv = jnp.minimum((nck - s * (CE // KB)) * (KB // 16), CE // 16)
                lax.fori_loop(0, nv, s1_vec, 0)
                return _c
            lax.fori_loop(0, nst, s1_stage, 0)

            def s2_stage(s, carry):
                pltpu.sync_copy(dst_h.at[pl.ds(a0 + s * CE, CE)], dst_v)
                pltpu.sync_copy(src_h.at[pl.ds(a0 + s * CE, CE)], src_v)
                ncb = jnp.minimum(nck - s * (CE // KB), CE // KB)

                def s2_chunk(ci, carry2):
                    off0 = ci * KB
                    for q in range(KB // 16):
                        idx_v[pl.ds(q * 16, 16)] = (
                            src_v[pl.ds(off0 + q * 16, 16)])
                    pltpu.async_copy(ft_h.at[idx_v], row_v, sem).wait()

                    def s2_vec(b, carry3):
                        off = off0 + b * 16
                        ge = a0 + s * CE + off + it
                        valid = (ge >= es) & (ge < ee)
                        kraw, sv = load_keys(off)
                        kl = jnp.where(valid, kraw - lo, DUMMY)
                        e = edge_logits(sv, kl)
                        emg = plsc.load_gather(em_v, [kl])
                        un = jnp.exp(e - emg)
                        un = jnp.where(valid, un, 0.0)

                        acc, denc, prevk = carry3
                        shifted = _shift_up(kl, 1, prevk)
                        isnew = (kl != shifted).astype(jnp.int32)
                        dsum = _seg_scan(kl, un, jnp.add)
                        dsum = dsum + jnp.where(kl == prevk, denc, 0.0)

                        def edge_body(j, acc_c):
                            accj = acc_c
                            w = _splat(un, j)
                            nsp = _splat(isnew, j) > 0
                            rsp = _splat(kl, j)
                            dsp = _splat(dsum, j)
                            rden = 1.0 / jnp.maximum(dsp, 1e-16)
                            ridx = jnp.full((16,), b * 16, jnp.int32) + j
                            new_acc = []
                            for k in range(D // 16):
                                rk = plsc.load_gather(row_v, [ridx, k * 16 + it])
                                a = jnp.where(nsp, w * rk, accj[k] + w * rk)
                                new_acc.append(a)
                                plsc.store_scatter(
                                    stg_v, [rsp * D + k * 16 + it], a * rden)
                            return tuple(new_acc)

                        acc = lax.fori_loop(0, 16, edge_body, acc)
                        prevk = _splat(kl, 15)
                        denc = _splat(dsum, 15)
                        return (acc, denc, prevk)

                    return lax.fori_loop(0, 3, s2_vec, carry2)

                return lax.fori_loop(0, ncb, s2_chunk, carry)

            acc0 = tuple(jnp.zeros((16,), jnp.float32) for _ in range(D // 16))
            den0 = jnp.zeros((16,), jnp.float32)
            prev0 = jnp.full((16,), -1, jnp.int32)
            lax.fori_loop(0, nst, s2_stage, (acc0, den0, prev0))

            pltpu.sync_copy(stg_v.at[pl.ds(0, NSR * D)],
                            out_h.at[pl.ds(lo * D, NSR * D)])
            return _

        lax.fori_loop(0, RPW, range_body, 0)

    return body(ft, el, er_p, dst_p, src_p, ebnd)


def _tc_layer1(feat, W1, resW1, alv, arv):
    BR = 1000

    def body(x_ref, w_ref, rw_ref, al_ref, ar_ref,
             ft_ref, res_ref, el_ref, er_ref):
        ft = jnp.dot(x_ref[...], w_ref[...],
                     preferred_element_type=jnp.float32)
        ft_ref[...] = ft
        res_ref[...] = jnp.dot(x_ref[...], rw_ref[...],
                               preferred_element_type=jnp.float32)
        el_ref[...] = jnp.dot(ft, al_ref[...],
                              preferred_element_type=jnp.float32)
        er_ref[...] = jnp.dot(ft, ar_ref[...],
                              preferred_element_type=jnp.float32)

    IN = feat.shape[1]
    return pl.pallas_call(
        body,
        grid=(N // BR,),
        in_specs=[
            pl.BlockSpec((BR, IN), lambda i: (i, 0)),
            pl.BlockSpec((IN, D), lambda i: (0, 0)),
            pl.BlockSpec((IN, D), lambda i: (0, 0)),
            pl.BlockSpec((D, 1), lambda i: (0, 0)),
            pl.BlockSpec((D, 1), lambda i: (0, 0)),
        ],
        out_specs=[
            pl.BlockSpec((BR, D), lambda i: (i, 0)),
            pl.BlockSpec((BR, D), lambda i: (i, 0)),
            pl.BlockSpec((BR, 1), lambda i: (i, 0)),
            pl.BlockSpec((BR, 1), lambda i: (i, 0)),
        ],
        out_shape=[
            jax.ShapeDtypeStruct((N, D), jnp.float32),
            jax.ShapeDtypeStruct((N, D), jnp.float32),
            jax.ShapeDtypeStruct((N, 1), jnp.float32),
            jax.ShapeDtypeStruct((N, 1), jnp.float32),
        ],
    )(feat, W1, resW1, alv, arv)


def _tc_layer_mid(rst, res_prev, b_prev, W, alv, arv):
    BR = 1000

    def body(rst_ref, rp_ref, b_ref, w_ref, al_ref, ar_ref,
             x_ref, ft_ref, el_ref, er_ref):
        x = jnp.maximum(rst_ref[...] + rp_ref[...] + b_ref[...], 0.0)
        x_ref[...] = x
        ft = jnp.dot(x, w_ref[...], preferred_element_type=jnp.float32)
        ft_ref[...] = ft
        el_ref[...] = jnp.dot(ft, al_ref[...],
                              preferred_element_type=jnp.float32)
        er_ref[...] = jnp.dot(ft, ar_ref[...],
                              preferred_element_type=jnp.float32)

    return pl.pallas_call(
        body,
        grid=(N // BR,),
        in_specs=[
            pl.BlockSpec((BR, D), lambda i: (i, 0)),
            pl.BlockSpec((BR, D), lambda i: (i, 0)),
            pl.BlockSpec((1, D), lambda i: (0, 0)),
            pl.BlockSpec((D, D), lambda i: (0, 0)),
            pl.BlockSpec((D, 1), lambda i: (0, 0)),
            pl.BlockSpec((D, 1), lambda i: (0, 0)),
        ],
        out_specs=[
            pl.BlockSpec((BR, D), lambda i: (i, 0)),
            pl.BlockSpec((BR, D), lambda i: (i, 0)),
            pl.BlockSpec((BR, 1), lambda i: (i, 0)),
            pl.BlockSpec((BR, 1), lambda i: (i, 0)),
        ],
        out_shape=[
            jax.ShapeDtypeStruct((N, D), jnp.float32),
            jax.ShapeDtypeStruct((N, D), jnp.float32),
            jax.ShapeDtypeStruct((N, 1), jnp.float32),
            jax.ShapeDtypeStruct((N, 1), jnp.float32),
        ],
    )(rst, res_prev, b_prev, W, alv, arv)


def _tc_final(rst, res_prev, b_prev):
    BR = 1000

    def body(rst_ref, rp_ref, b_ref, o_ref):
        o_ref[...] = jnp.maximum(rst_ref[...] + rp_ref[...] + b_ref[...], 0.0)

    return pl.pallas_call(
        body,
        grid=(N // BR,),
        in_specs=[
            pl.BlockSpec((BR, D), lambda i: (i, 0)),
            pl.BlockSpec((BR, D), lambda i: (i, 0)),
            pl.BlockSpec((1, D), lambda i: (0, 0)),
        ],
        out_specs=pl.BlockSpec((BR, D), lambda i: (i, 0)),
        out_shape=jax.ShapeDtypeStruct((N, D), jnp.float32),
    )(rst, res_prev, b_prev)


def _prep_edges(edge_index):
    src = edge_index[0].astype(jnp.int32)
    dst = edge_index[1].astype(jnp.int32)
    dst_s, src_s = lax.sort((dst, src), num_keys=1)
    node_bounds = jnp.minimum(jnp.arange(R + 1, dtype=jnp.int32) * NSR, N)
    ebnd = jnp.searchsorted(dst_s, node_bounds, side="left").astype(jnp.int32)
    ebnd = jnp.pad(ebnd, (0, 144 - (R + 1)), constant_values=E)
    dst_p = jnp.pad(dst_s, (0, PADE - E), constant_values=N - 1)
    src_p = jnp.pad(src_s, (0, PADE - E), constant_values=0)
    return dst_p, src_p, ebnd


def _pad_er(er):
    return jnp.pad(er.reshape(N), (0, 10256 - N))


def kernel(feat, edge_index, W1, al1, ar1, b1, resW1, W2, al2, ar2, b2,
           W3, al3, ar3, b3, W4, al4, ar4, b4):
    dst_p, src_p, ebnd = _prep_edges(edge_index)

    def agg(ft, el, er):
        rst = _sc_aggregate(ft, el.reshape(N), _pad_er(er),
                            dst_p, src_p, ebnd)
        return rst.reshape(NPAD, D)[:N]

    ft1, res1, el1, er1 = _tc_layer1(
        feat, W1, resW1, al1.reshape(D, 1), ar1.reshape(D, 1))
    rst1 = agg(ft1, el1, er1)

    x1, ft2, el2, er2 = _tc_layer_mid(
        rst1, res1, b1.reshape(1, D), W2, al2.reshape(D, 1), ar2.reshape(D, 1))
    rst2 = agg(ft2, el2, er2)

    x2, ft3, el3, er3 = _tc_layer_mid(
        rst2, x1, b2.reshape(1, D), W3, al3.reshape(D, 1), ar3.reshape(D, 1))
    rst3 = agg(ft3, el3, er3)

    x3, ft4, el4, er4 = _tc_layer_mid(
        rst3, x2, b3.reshape(1, D), W4, al4.reshape(D, 1), ar4.reshape(D, 1))
    rst4 = agg(ft4, el4, er4)

    return _tc_final(rst4, x3, b4.reshape(1, D))


# speedup vs baseline: 3.6042x; 1.0717x over previous
import functools

import jax
import jax.numpy as jnp
from jax import lax
from jax.experimental import pallas as pl
from jax.experimental.pallas import tpu as pltpu
from jax.experimental.pallas import tpu_sc as plsc

N = 10000
E = 160000
D = 512

NW = 32
NSR = 80
R = 128
RPW = R // NW
NPAD = R * NSR
DUMMY = NSR

KB = 48
CPS = 44
CE = KB * CPS
PADE = E + CE + 16

_I16 = lambda: lax.iota(jnp.int32, 16)


def _dg(v, idx):
    return jnp.take_along_axis(v, idx, axis=0, mode="promise_in_bounds")


def _splat(v, j):
    return _dg(v, jnp.full((16,), j, dtype=jnp.int32))


def _sget_i32(ref, j):
    base = (j // 16) * 16
    v = ref[pl.ds(base, 16)]
    rot = _dg(v, (_I16() + (j - base)) & 15)
    return rot[0]


def _shift_up(v, d, fill):
    it = _I16()
    s = _dg(v, jnp.maximum(it - d, 0))
    return jnp.where(it >= d, s, fill)


def _seg_scan(key, val, op):
    it = _I16()
    for d in (1, 2, 4, 8):
        idx = jnp.maximum(it - d, 0)
        sk = _dg(key, idx)
        sv = _dg(val, idx)
        cond = (sk == key) & (it >= d)
        val = jnp.where(cond, op(val, sv), val)
    return val


def _sc_aggregate(ft, el, er_p, dst_p, src_p, ebnd):
    mesh = plsc.VectorSubcoreMesh(core_axis_name="c", subcore_axis_name="s",
                                  num_cores=2, num_subcores=16)

    @functools.partial(
        pl.kernel,
        out_type=jax.ShapeDtypeStruct((NPAD * D,), jnp.float32),
        mesh=mesh,
        scratch_types=[
            pltpu.VMEM((N,), jnp.float32),
            pltpu.VMEM((96,), jnp.float32),
            pltpu.VMEM((96,), jnp.float32),
            pltpu.VMEM((144,), jnp.int32),
            pltpu.VMEM((CE,), jnp.int32),
            pltpu.VMEM((CE + KB,), jnp.int32),
            pltpu.VMEM((KB,), jnp.int32),
            pltpu.VMEM((KB,), jnp.int32),
            pltpu.VMEM((KB, D), jnp.float32),
            pltpu.VMEM((KB, D), jnp.float32),
            pltpu.VMEM(((NSR + 1) * D,), jnp.float32),
            pltpu.SemaphoreType.DMA,
            pltpu.SemaphoreType.DMA,
        ],
        compiler_params=pltpu.CompilerParams(needs_layout_passes=False,
                                             use_tc_tiling_on_sc=False),
    )
    def body(ft_h, el_h, er_h, dst_h, src_h, eb_h, out_h,
             el_v, er_v, em_v, eb_v, dst_v, src_v, idx0_v, idx1_v,
             row0_v, row1_v, stg_v, sem0, sem1):
        wid = lax.axis_index("s") * 2 + lax.axis_index("c")
        pltpu.sync_copy(el_h, el_v)
        pltpu.sync_copy(eb_h, eb_v)
        it = _I16()
        for q in range(KB // 16):
            src_v[pl.ds(CE + q * 16, 16)] = jnp.zeros((16,), jnp.int32)

        def range_body(r_i, _):
            r_glob = wid * RPW + r_i
            lo = r_glob * NSR
            es = _sget_i32(eb_v, r_glob)
            ee = _sget_i32(eb_v, r_glob + 1)
            a0 = (es // 8) * 8
            nedge = ee - es
            nck = (es - a0 + nedge + KB - 1) // KB
            nst = (nck + (CE // KB) - 1) // (CE // KB)

            pltpu.sync_copy(er_h.at[pl.ds(lo, 96)], er_v)

            def init_em(i, _c):
                em_v[pl.ds(i * 16, 16)] = jnp.full((16,), -3.4e38, jnp.float32)
                return _c
            lax.fori_loop(0, 6, init_em, 0)

            def init_stg(i, _c):
                stg_v[pl.ds(i * 16, 16)] = jnp.zeros((16,), jnp.float32)
                return _c
            lax.fori_loop(0, (NSR + 1) * D // 16, init_stg, 0)

            def load_keys(off):
                kraw = dst_v[pl.ds(off, 16)]
                sv = src_v[pl.ds(off, 16)]
                return kraw, sv

            def edge_logits(sv, kl):
                elg = plsc.load_gather(el_v, [sv])
                erg = plsc.load_gather(er_v, [kl])
                e = elg + erg
                return jnp.where(e > 0, e, 0.2 * e)

            def s1_stage(s, _c):
                pltpu.sync_copy(dst_h.at[pl.ds(a0 + s * CE, CE)], dst_v)
                pltpu.sync_copy(src_h.at[pl.ds(a0 + s * CE, CE)],
                                src_v.at[pl.ds(0, CE)])

                def s1_vec(v, _c2):
                    off = v * 16
                    ge = a0 + s * CE + off + it
                    valid = (ge >= es) & (ge < ee)
                    kraw, sv = load_keys(off)
                    kl = jnp.where(valid, kraw - lo, DUMMY)
                    e = edge_logits(sv, kl)
                    mx = _seg_scan(kl, e, jnp.maximum)
                    knext = _dg(kl, jnp.minimum(it + 1, 15))
                    endm = (kl != knext) | (it == 15)
                    cur = plsc.load_gather(em_v, [kl])
                    plsc.store_scatter(em_v, [kl], jnp.maximum(mx, cur),
                                       mask=endm)
                    return _c2

                nv = jnp.minimum((nck - s * (CE // KB)) * (KB // 16), CE // 16)
                lax.fori_loop(0, nv, s1_vec, 0)
                return _c
            lax.fori_loop(0, nst, s1_stage, 0)

            def s2_stage(s, carry):
                pltpu.sync_copy(dst_h.at[pl.ds(a0 + s * CE, CE)],
                                dst_v.at[pl.ds(0, CE)])
                pltpu.sync_copy(src_h.at[pl.ds(a0 + s * CE, CE)],
                                src_v.at[pl.ds(0, CE)])
                ncb = jnp.minimum(nck - s * CPS, CPS)
                nc2 = 2 * ((ncb + 1) // 2)

                def build_and_start(ci, idx_v, row_v, sem):
                    off0 = ci * KB
                    for q in range(KB // 16):
                        idx_v[pl.ds(q * 16, 16)] = (
                            src_v[pl.ds(off0 + q * 16, 16)])
                    pltpu.async_copy(ft_h.at[idx_v], row_v, sem)

                def s2_chunk(ci, row_v, carry2):
                    off0 = ci * KB

                    def s2_vec(b, carry3):
                        off = off0 + b * 16
                        ge = a0 + s * CE + off + it
                        valid = (ge >= es) & (ge < ee)
                        kraw, sv = load_keys(off)
                        kl = jnp.where(valid, kraw - lo, DUMMY)
                        e = edge_logits(sv, kl)
                        emg = plsc.load_gather(em_v, [kl])
                        un = jnp.exp(e - emg)
                        un = jnp.where(valid, un, 0.0)

                        acc, denc, prevk = carry3
                        shifted = _shift_up(kl, 1, prevk)
                        isnew = (kl != shifted).astype(jnp.int32)
                        dsum = _seg_scan(kl, un, jnp.add)
                        dsum = dsum + jnp.where(kl == prevk, denc, 0.0)

                        def edge_body(j, acc_c):
                            accj = acc_c
                            w = _splat(un, j)
                            nsp = _splat(isnew, j) > 0
                            rsp = _splat(kl, j)
                            dsp = _splat(dsum, j)
                            rden = 1.0 / jnp.maximum(dsp, 1e-16)
                            ridx = jnp.full((16,), b * 16, jnp.int32) + j
                            new_acc = []
                            for k in range(D // 16):
                                rk = plsc.load_gather(row_v, [ridx, k * 16 + it])
                                a = jnp.where(nsp, w * rk, accj[k] + w * rk)
                                new_acc.append(a)
                                plsc.store_scatter(
                                    stg_v, [rsp * D + k * 16 + it], a * rden)
                            return tuple(new_acc)

                        acc = lax.fori_loop(0, 16, edge_body, acc)
                        prevk = _splat(kl, 15)
                        denc = _splat(dsum, 15)
                        return (acc, denc, prevk)

                    return lax.fori_loop(0, 3, s2_vec, carry2)

                def wait_rows(idx_v, row_v, sem):
                    pltpu.make_async_copy(ft_h.at[idx_v], row_v, sem).wait()

                def pair_body(p, carry2):
                    c0 = 2 * p
                    build_and_start(c0 + 1, idx1_v, row1_v, sem1)
                    wait_rows(idx0_v, row0_v, sem0)
                    carry2 = s2_chunk(c0, row0_v, carry2)

                    @pl.when(c0 + 2 < nc2)
                    def _():
                        build_and_start(c0 + 2, idx0_v, row0_v, sem0)

                    wait_rows(idx1_v, row1_v, sem1)
                    return s2_chunk(c0 + 1, row1_v, carry2)

                @pl.when(nc2 > 0)
                def _():
                    build_and_start(0, idx0_v, row0_v, sem0)

                return lax.fori_loop(0, nc2 // 2, pair_body, carry)

            acc0 = tuple(jnp.zeros((16,), jnp.float32) for _ in range(D // 16))
            den0 = jnp.zeros((16,), jnp.float32)
            prev0 = jnp.full((16,), -1, jnp.int32)
            lax.fori_loop(0, nst, s2_stage, (acc0, den0, prev0))

            pltpu.sync_copy(stg_v.at[pl.ds(0, NSR * D)],
                            out_h.at[pl.ds(lo * D, NSR * D)])
            return _

        lax.fori_loop(0, RPW, range_body, 0)

    return body(ft, el, er_p, dst_p, src_p, ebnd)


def _tc_layer1(feat, W1, resW1, alv, arv):
    BR = 1000

    def body(x_ref, w_ref, rw_ref, al_ref, ar_ref,
             ft_ref, res_ref, el_ref, er_ref):
        ft = jnp.dot(x_ref[...], w_ref[...],
                     preferred_element_type=jnp.float32)
        ft_ref[...] = ft
        res_ref[...] = jnp.dot(x_ref[...], rw_ref[...],
                               preferred_element_type=jnp.float32)
        el_ref[...] = jnp.dot(ft, al_ref[...],
                              preferred_element_type=jnp.float32)
        er_ref[...] = jnp.dot(ft, ar_ref[...],
                              preferred_element_type=jnp.float32)

    IN = feat.shape[1]
    return pl.pallas_call(
        body,
        grid=(N // BR,),
        in_specs=[
            pl.BlockSpec((BR, IN), lambda i: (i, 0)),
            pl.BlockSpec((IN, D), lambda i: (0, 0)),
            pl.BlockSpec((IN, D), lambda i: (0, 0)),
            pl.BlockSpec((D, 1), lambda i: (0, 0)),
            pl.BlockSpec((D, 1), lambda i: (0, 0)),
        ],
        out_specs=[
            pl.BlockSpec((BR, D), lambda i: (i, 0)),
            pl.BlockSpec((BR, D), lambda i: (i, 0)),
            pl.BlockSpec((BR, 1), lambda i: (i, 0)),
            pl.BlockSpec((BR, 1), lambda i: (i, 0)),
        ],
        out_shape=[
            jax.ShapeDtypeStruct((N, D), jnp.float32),
            jax.ShapeDtypeStruct((N, D), jnp.float32),
            jax.ShapeDtypeStruct((N, 1), jnp.float32),
            jax.ShapeDtypeStruct((N, 1), jnp.float32),
        ],
    )(feat, W1, resW1, alv, arv)


def _tc_layer_mid(rst, res_prev, b_prev, W, alv, arv):
    BR = 1000

    def body(rst_ref, rp_ref, b_ref, w_ref, al_ref, ar_ref,
             x_ref, ft_ref, el_ref, er_ref):
        x = jnp.maximum(rst_ref[...] + rp_ref[...] + b_ref[...], 0.0)
        x_ref[...] = x
        ft = jnp.dot(x, w_ref[...], preferred_element_type=jnp.float32)
        ft_ref[...] = ft
        el_ref[...] = jnp.dot(ft, al_ref[...],
                              preferred_element_type=jnp.float32)
        er_ref[...] = jnp.dot(ft, ar_ref[...],
                              preferred_element_type=jnp.float32)

    return pl.pallas_call(
        body,
        grid=(N // BR,),
        in_specs=[
            pl.BlockSpec((BR, D), lambda i: (i, 0)),
            pl.BlockSpec((BR, D), lambda i: (i, 0)),
            pl.BlockSpec((1, D), lambda i: (0, 0)),
            pl.BlockSpec((D, D), lambda i: (0, 0)),
            pl.BlockSpec((D, 1), lambda i: (0, 0)),
            pl.BlockSpec((D, 1), lambda i: (0, 0)),
        ],
        out_specs=[
            pl.BlockSpec((BR, D), lambda i: (i, 0)),
            pl.BlockSpec((BR, D), lambda i: (i, 0)),
            pl.BlockSpec((BR, 1), lambda i: (i, 0)),
            pl.BlockSpec((BR, 1), lambda i: (i, 0)),
        ],
        out_shape=[
            jax.ShapeDtypeStruct((N, D), jnp.float32),
            jax.ShapeDtypeStruct((N, D), jnp.float32),
            jax.ShapeDtypeStruct((N, 1), jnp.float32),
            jax.ShapeDtypeStruct((N, 1), jnp.float32),
        ],
    )(rst, res_prev, b_prev, W, alv, arv)


def _tc_final(rst, res_prev, b_prev):
    BR = 1000

    def body(rst_ref, rp_ref, b_ref, o_ref):
        o_ref[...] = jnp.maximum(rst_ref[...] + rp_ref[...] + b_ref[...], 0.0)

    return pl.pallas_call(
        body,
        grid=(N // BR,),
        in_specs=[
            pl.BlockSpec((BR, D), lambda i: (i, 0)),
            pl.BlockSpec((BR, D), lambda i: (i, 0)),
            pl.BlockSpec((1, D), lambda i: (0, 0)),
        ],
        out_specs=pl.BlockSpec((BR, D), lambda i: (i, 0)),
        out_shape=jax.ShapeDtypeStruct((N, D), jnp.float32),
    )(rst, res_prev, b_prev)


def _prep_edges(edge_index):
    src = edge_index[0].astype(jnp.int32)
    dst = edge_index[1].astype(jnp.int32)
    dst_s, src_s = lax.sort((dst, src), num_keys=1)
    node_bounds = jnp.minimum(jnp.arange(R + 1, dtype=jnp.int32) * NSR, N)
    ebnd = jnp.searchsorted(dst_s, node_bounds, side="left").astype(jnp.int32)
    ebnd = jnp.pad(ebnd, (0, 144 - (R + 1)), constant_values=E)
    dst_p = jnp.pad(dst_s, (0, PADE - E), constant_values=N - 1)
    src_p = jnp.pad(src_s, (0, PADE - E), constant_values=0)
    return dst_p, src_p, ebnd


def _pad_er(er):
    return jnp.pad(er.reshape(N), (0, 10256 - N))


def kernel(feat, edge_index, W1, al1, ar1, b1, resW1, W2, al2, ar2, b2,
           W3, al3, ar3, b3, W4, al4, ar4, b4):
    dst_p, src_p, ebnd = _prep_edges(edge_index)

    def agg(ft, el, er):
        rst = _sc_aggregate(ft, el.reshape(N), _pad_er(er),
                            dst_p, src_p, ebnd)
        return rst.reshape(NPAD, D)[:N]

    ft1, res1, el1, er1 = _tc_layer1(
        feat, W1, resW1, al1.reshape(D, 1), ar1.reshape(D, 1))
    rst1 = agg(ft1, el1, er1)

    x1, ft2, el2, er2 = _tc_layer_mid(
        rst1, res1, b1.reshape(1, D), W2, al2.reshape(D, 1), ar2.reshape(D, 1))
    rst2 = agg(ft2, el2, er2)

    x2, ft3, el3, er3 = _tc_layer_mid(
        rst2, x1, b2.reshape(1, D), W3, al3.reshape(D, 1), ar3.reshape(D, 1))
    rst3 = agg(ft3, el3, er3)

    x3, ft4, el4, er4 = _tc_layer_mid(
        rst3, x2, b3.reshape(1, D), W4, al4.reshape(D, 1), ar4.reshape(D, 1))
    rst4 = agg(ft4, el4, er4)

    return _tc_final(rst4, x3, b4.reshape(1, D))


# speedup vs baseline: 9.5970x; 2.6627x over previous
import functools

import jax
import jax.numpy as jnp
from jax import lax
from jax.experimental import pallas as pl
from jax.experimental.pallas import tpu as pltpu
from jax.experimental.pallas import tpu_sc as plsc

N = 10000
E = 160000
D = 512

NW = 32
NSR = 80
R = 128
RPW = R // NW
NPAD = R * NSR
DUMMY = NSR

KB = 48
CPS = 44
CE = KB * CPS
PADE = E + CE + 16

_I16 = lambda: lax.iota(jnp.int32, 16)


def _dg(v, idx):
    return jnp.take_along_axis(v, idx, axis=0, mode="promise_in_bounds")


def _splat(v, j):
    return _dg(v, jnp.full((16,), j, dtype=jnp.int32))


def _sget_i32(ref, j):
    base = (j // 16) * 16
    v = ref[pl.ds(base, 16)]
    rot = _dg(v, (_I16() + (j - base)) & 15)
    return rot[0]


def _shift_up(v, d, fill):
    it = _I16()
    s = _dg(v, jnp.maximum(it - d, 0))
    return jnp.where(it >= d, s, fill)


def _seg_scan(key, val, op):
    it = _I16()
    for d in (1, 2, 4, 8):
        idx = jnp.maximum(it - d, 0)
        sk = _dg(key, idx)
        sv = _dg(val, idx)
        cond = (sk == key) & (it >= d)
        val = jnp.where(cond, op(val, sv), val)
    return val


def _sc_aggregate(ft, el, er_p, dst_p, src_p, ebnd):
    mesh = plsc.VectorSubcoreMesh(core_axis_name="c", subcore_axis_name="s",
                                  num_cores=2, num_subcores=16)

    @functools.partial(
        pl.kernel,
        out_type=jax.ShapeDtypeStruct((NPAD * D,), jnp.float32),
        mesh=mesh,
        scratch_types=[
            pltpu.VMEM((N,), jnp.float32),
            pltpu.VMEM((96,), jnp.float32),
            pltpu.VMEM((96,), jnp.float32),
            pltpu.VMEM((144,), jnp.int32),
            pltpu.VMEM((CE,), jnp.int32),
            pltpu.VMEM((CE + KB,), jnp.int32),
            pltpu.VMEM((KB,), jnp.int32),
            pltpu.VMEM((KB,), jnp.int32),
            pltpu.VMEM((KB, D), jnp.float32),
            pltpu.VMEM((KB, D), jnp.float32),
            pltpu.VMEM(((NSR + 1) * D,), jnp.float32),
            pltpu.SemaphoreType.DMA,
            pltpu.SemaphoreType.DMA,
        ],
        compiler_params=pltpu.CompilerParams(needs_layout_passes=False,
                                             use_tc_tiling_on_sc=False),
    )
    def body(ft_h, el_h, er_h, dst_h, src_h, eb_h, out_h,
             el_v, er_v, em_v, eb_v, dst_v, src_v, idx0_v, idx1_v,
             row0_v, row1_v, stg_v, sem0, sem1):
        wid = lax.axis_index("s") * 2 + lax.axis_index("c")
        pltpu.sync_copy(el_h, el_v)
        pltpu.sync_copy(eb_h, eb_v)
        it = _I16()
        for q in range(KB // 16):
            src_v[pl.ds(CE + q * 16, 16)] = jnp.zeros((16,), jnp.int32)

        def range_body(r_i, _):
            r_glob = wid * RPW + r_i
            lo = r_glob * NSR
            es = _sget_i32(eb_v, r_glob)
            ee = _sget_i32(eb_v, r_glob + 1)
            a0 = (es // 8) * 8
            nedge = ee - es
            nck = (es - a0 + nedge + KB - 1) // KB
            nst = (nck + (CE // KB) - 1) // (CE // KB)

            pltpu.sync_copy(er_h.at[pl.ds(lo, 96)], er_v)

            def init_em(i, _c):
                em_v[pl.ds(i * 16, 16)] = jnp.full((16,), -3.4e38, jnp.float32)
                return _c
            lax.fori_loop(0, 6, init_em, 0)

            def init_stg(i, _c):
                stg_v[pl.ds(i * 16, 16)] = jnp.zeros((16,), jnp.float32)
                return _c
            lax.fori_loop(0, (NSR + 1) * D // 16, init_stg, 0)

            def load_keys(off):
                kraw = dst_v[pl.ds(off, 16)]
                sv = src_v[pl.ds(off, 16)]
                return kraw, sv

            def edge_logits(sv, kl):
                elg = plsc.load_gather(el_v, [sv])
                erg = plsc.load_gather(er_v, [kl])
                e = elg + erg
                return jnp.where(e > 0, e, 0.2 * e)

            def s1_stage(s, _c):
                pltpu.sync_copy(dst_h.at[pl.ds(a0 + s * CE, CE)], dst_v)
                pltpu.sync_copy(src_h.at[pl.ds(a0 + s * CE, CE)],
                                src_v.at[pl.ds(0, CE)])

                def s1_vec(v, _c2):
                    off = v * 16
                    ge = a0 + s * CE + off + it
                    valid = (ge >= es) & (ge < ee)
                    kraw, sv = load_keys(off)
                    kl = jnp.where(valid, kraw - lo, DUMMY)
                    e = edge_logits(sv, kl)
                    mx = _seg_scan(kl, e, jnp.maximum)
                    knext = _dg(kl, jnp.minimum(it + 1, 15))
                    endm = (kl != knext) | (it == 15)
                    cur = plsc.load_gather(em_v, [kl])
                    plsc.store_scatter(em_v, [kl], jnp.maximum(mx, cur),
                                       mask=endm)
                    return _c2

                nv = jnp.minimum((nck - s * (CE // KB)) * (KB // 16), CE // 16)
                lax.fori_loop(0, nv, s1_vec, 0)
                return _c
            lax.fori_loop(0, nst, s1_stage, 0)

            def s2_stage(s, carry):
                pltpu.sync_copy(dst_h.at[pl.ds(a0 + s * CE, CE)],
                                dst_v.at[pl.ds(0, CE)])
                pltpu.sync_copy(src_h.at[pl.ds(a0 + s * CE, CE)],
                                src_v.at[pl.ds(0, CE)])
                ncb = jnp.minimum(nck - s * CPS, CPS)
                nc2 = 2 * ((ncb + 1) // 2)

                def build_and_start(ci, idx_v, row_v, sem):
                    off0 = ci * KB
                    for q in range(KB // 16):
                        idx_v[pl.ds(q * 16, 16)] = (
                            src_v[pl.ds(off0 + q * 16, 16)])
                    pltpu.async_copy(ft_h.at[idx_v], row_v, sem)

                def s2_chunk(ci, row_v, carry2):
                    off0 = ci * KB

                    def s2_vec(b, carry3):
                        off = off0 + b * 16
                        ge = a0 + s * CE + off + it
                        valid = (ge >= es) & (ge < ee)
                        kraw, sv = load_keys(off)
                        kl = jnp.where(valid, kraw - lo, DUMMY)
                        e = edge_logits(sv, kl)
                        emg = plsc.load_gather(em_v, [kl])
                        un = jnp.exp(e - emg)
                        un = jnp.where(valid, un, 0.0)

                        acc, denc, prevk = carry3
                        shifted = _shift_up(kl, 1, prevk)
                        isnew = (kl != shifted).astype(jnp.int32)
                        dsum = _seg_scan(kl, un, jnp.add)
                        dsum = dsum + jnp.where(kl == prevk, denc, 0.0)
                        knext = _dg(kl, jnp.minimum(it + 1, 15))
                        endm = ((kl != knext) | (it == 15)).astype(jnp.int32)

                        def edge_body(j, acc_c):
                            accj = acc_c
                            w = _splat(un, j)
                            nsp = _splat(isnew, j) > 0
                            ridx = jnp.full((16,), b * 16, jnp.int32) + j
                            new_acc = []
                            for k in range(D // 16):
                                rk = plsc.load_gather(row_v, [ridx, k * 16 + it])
                                a = jnp.where(nsp, w * rk, accj[k] + w * rk)
                                new_acc.append(a)
                            endrot = _dg(endm, (it + j) & 15)

                            @pl.when(endrot[0] > 0)
                            def _():
                                rsp = _splat(kl, j)
                                dsp = _splat(dsum, j)
                                rden = 1.0 / jnp.maximum(dsp, 1e-16)
                                for k in range(D // 16):
                                    plsc.store_scatter(
                                        stg_v, [rsp * D + k * 16 + it],
                                        new_acc[k] * rden)

                            return tuple(new_acc)

                        acc = lax.fori_loop(0, 16, edge_body, acc)
                        prevk = _splat(kl, 15)
                        denc = _splat(dsum, 15)
                        return (acc, denc, prevk)

                    return lax.fori_loop(0, 3, s2_vec, carry2)

                def wait_rows(idx_v, row_v, sem):
                    pltpu.make_async_copy(ft_h.at[idx_v], row_v, sem).wait()

                def pair_body(p, carry2):
                    c0 = 2 * p
                    build_and_start(c0 + 1, idx1_v, row1_v, sem1)
                    wait_rows(idx0_v, row0_v, sem0)
                    carry2 = s2_chunk(c0, row0_v, carry2)

                    @pl.when(c0 + 2 < nc2)
                    def _():
                        build_and_start(c0 + 2, idx0_v, row0_v, sem0)

                    wait_rows(idx1_v, row1_v, sem1)
                    return s2_chunk(c0 + 1, row1_v, carry2)

                @pl.when(nc2 > 0)
                def _():
                    build_and_start(0, idx0_v, row0_v, sem0)

                return lax.fori_loop(0, nc2 // 2, pair_body, carry)

            acc0 = tuple(jnp.zeros((16,), jnp.float32) for _ in range(D // 16))
            den0 = jnp.zeros((16,), jnp.float32)
            prev0 = jnp.full((16,), -1, jnp.int32)
            lax.fori_loop(0, nst, s2_stage, (acc0, den0, prev0))

            pltpu.sync_copy(stg_v.at[pl.ds(0, NSR * D)],
                            out_h.at[pl.ds(lo * D, NSR * D)])
            return _

        lax.fori_loop(0, RPW, range_body, 0)

    return body(ft, el, er_p, dst_p, src_p, ebnd)


def _tc_layer1(feat, W1, resW1, alv, arv):
    BR = 1000

    def body(x_ref, w_ref, rw_ref, al_ref, ar_ref,
             ft_ref, res_ref, el_ref, er_ref):
        ft = jnp.dot(x_ref[...], w_ref[...],
                     preferred_element_type=jnp.float32)
        ft_ref[...] = ft
        res_ref[...] = jnp.dot(x_ref[...], rw_ref[...],
                               preferred_element_type=jnp.float32)
        el_ref[...] = jnp.dot(ft, al_ref[...],
                              preferred_element_type=jnp.float32)
        er_ref[...] = jnp.dot(ft, ar_ref[...],
                              preferred_element_type=jnp.float32)

    IN = feat.shape[1]
    return pl.pallas_call(
        body,
        grid=(N // BR,),
        in_specs=[
            pl.BlockSpec((BR, IN), lambda i: (i, 0)),
            pl.BlockSpec((IN, D), lambda i: (0, 0)),
            pl.BlockSpec((IN, D), lambda i: (0, 0)),
            pl.BlockSpec((D, 1), lambda i: (0, 0)),
            pl.BlockSpec((D, 1), lambda i: (0, 0)),
        ],
        out_specs=[
            pl.BlockSpec((BR, D), lambda i: (i, 0)),
            pl.BlockSpec((BR, D), lambda i: (i, 0)),
            pl.BlockSpec((BR, 1), lambda i: (i, 0)),
            pl.BlockSpec((BR, 1), lambda i: (i, 0)),
        ],
        out_shape=[
            jax.ShapeDtypeStruct((N, D), jnp.float32),
            jax.ShapeDtypeStruct((N, D), jnp.float32),
            jax.ShapeDtypeStruct((N, 1), jnp.float32),
            jax.ShapeDtypeStruct((N, 1), jnp.float32),
        ],
    )(feat, W1, resW1, alv, arv)


def _tc_layer_mid(rst, res_prev, b_prev, W, alv, arv):
    BR = 1000

    def body(rst_ref, rp_ref, b_ref, w_ref, al_ref, ar_ref,
             x_ref, ft_ref, el_ref, er_ref):
        x = jnp.maximum(rst_ref[...] + rp_ref[...] + b_ref[...], 0.0)
        x_ref[...] = x
        ft = jnp.dot(x, w_ref[...], preferred_element_type=jnp.float32)
        ft_ref[...] = ft
        el_ref[...] = jnp.dot(ft, al_ref[...],
                              preferred_element_type=jnp.float32)
        er_ref[...] = jnp.dot(ft, ar_ref[...],
                              preferred_element_type=jnp.float32)

    return pl.pallas_call(
        body,
        grid=(N // BR,),
        in_specs=[
            pl.BlockSpec((BR, D), lambda i: (i, 0)),
            pl.BlockSpec((BR, D), lambda i: (i, 0)),
            pl.BlockSpec((1, D), lambda i: (0, 0)),
            pl.BlockSpec((D, D), lambda i: (0, 0)),
            pl.BlockSpec((D, 1), lambda i: (0, 0)),
            pl.BlockSpec((D, 1), lambda i: (0, 0)),
        ],
        out_specs=[
            pl.BlockSpec((BR, D), lambda i: (i, 0)),
            pl.BlockSpec((BR, D), lambda i: (i, 0)),
            pl.BlockSpec((BR, 1), lambda i: (i, 0)),
            pl.BlockSpec((BR, 1), lambda i: (i, 0)),
        ],
        out_shape=[
            jax.ShapeDtypeStruct((N, D), jnp.float32),
            jax.ShapeDtypeStruct((N, D), jnp.float32),
            jax.ShapeDtypeStruct((N, 1), jnp.float32),
            jax.ShapeDtypeStruct((N, 1), jnp.float32),
        ],
    )(rst, res_prev, b_prev, W, alv, arv)


def _tc_final(rst, res_prev, b_prev):
    BR = 1000

    def body(rst_ref, rp_ref, b_ref, o_ref):
        o_ref[...] = jnp.maximum(rst_ref[...] + rp_ref[...] + b_ref[...], 0.0)

    return pl.pallas_call(
        body,
        grid=(N // BR,),
        in_specs=[
            pl.BlockSpec((BR, D), lambda i: (i, 0)),
            pl.BlockSpec((BR, D), lambda i: (i, 0)),
            pl.BlockSpec((1, D), lambda i: (0, 0)),
        ],
        out_specs=pl.BlockSpec((BR, D), lambda i: (i, 0)),
        out_shape=jax.ShapeDtypeStruct((N, D), jnp.float32),
    )(rst, res_prev, b_prev)


def _prep_edges(edge_index):
    src = edge_index[0].astype(jnp.int32)
    dst = edge_index[1].astype(jnp.int32)
    dst_s, src_s = lax.sort((dst, src), num_keys=1)
    node_bounds = jnp.minimum(jnp.arange(R + 1, dtype=jnp.int32) * NSR, N)
    ebnd = jnp.searchsorted(dst_s, node_bounds, side="left").astype(jnp.int32)
    ebnd = jnp.pad(ebnd, (0, 144 - (R + 1)), constant_values=E)
    dst_p = jnp.pad(dst_s, (0, PADE - E), constant_values=N - 1)
    src_p = jnp.pad(src_s, (0, PADE - E), constant_values=0)
    return dst_p, src_p, ebnd


def _pad_er(er):
    return jnp.pad(er.reshape(N), (0, 10256 - N))


def kernel(feat, edge_index, W1, al1, ar1, b1, resW1, W2, al2, ar2, b2,
           W3, al3, ar3, b3, W4, al4, ar4, b4):
    dst_p, src_p, ebnd = _prep_edges(edge_index)

    def agg(ft, el, er):
        rst = _sc_aggregate(ft, el.reshape(N), _pad_er(er),
                            dst_p, src_p, ebnd)
        return rst.reshape(NPAD, D)[:N]

    ft1, res1, el1, er1 = _tc_layer1(
        feat, W1, resW1, al1.reshape(D, 1), ar1.reshape(D, 1))
    rst1 = agg(ft1, el1, er1)

    x1, ft2, el2, er2 = _tc_layer_mid(
        rst1, res1, b1.reshape(1, D), W2, al2.reshape(D, 1), ar2.reshape(D, 1))
    rst2 = agg(ft2, el2, er2)

    x2, ft3, el3, er3 = _tc_layer_mid(
        rst2, x1, b2.reshape(1, D), W3, al3.reshape(D, 1), ar3.reshape(D, 1))
    rst3 = agg(ft3, el3, er3)

    x3, ft4, el4, er4 = _tc_layer_mid(
        rst3, x2, b3.reshape(1, D), W4, al4.reshape(D, 1), ar4.reshape(D, 1))
    rst4 = agg(ft4, el4, er4)

    return _tc_final(rst4, x3, b4.reshape(1, D))


# speedup vs baseline: 13.1345x; 1.3686x over previous
import functools

import jax
import jax.numpy as jnp
from jax import lax
from jax.experimental import pallas as pl
from jax.experimental.pallas import tpu as pltpu
from jax.experimental.pallas import tpu_sc as plsc

N = 10000
E = 160000
D = 512

NW = 32
NSR = 80
R = 128
RPW = R // NW
NPAD = R * NSR
DUMMY = NSR

KB = 48
CPS = 44
CE = KB * CPS
PADE = E + CE + 16

_I16 = lambda: lax.iota(jnp.int32, 16)


def _dg(v, idx):
    return jnp.take_along_axis(v, idx, axis=0, mode="promise_in_bounds")


def _splat(v, j):
    return _dg(v, jnp.full((16,), j, dtype=jnp.int32))


def _sget_i32(ref, j):
    base = (j // 16) * 16
    v = ref[pl.ds(base, 16)]
    rot = _dg(v, (_I16() + (j - base)) & 15)
    return rot[0]


def _shift_up(v, d, fill):
    it = _I16()
    s = _dg(v, jnp.maximum(it - d, 0))
    return jnp.where(it >= d, s, fill)


def _seg_scan(key, val, op):
    it = _I16()
    for d in (1, 2, 4, 8):
        idx = jnp.maximum(it - d, 0)
        sk = _dg(key, idx)
        sv = _dg(val, idx)
        cond = (sk == key) & (it >= d)
        val = jnp.where(cond, op(val, sv), val)
    return val


def _sc_aggregate(ft, el, er_p, dst_p, src_p, ebnd):
    mesh = plsc.VectorSubcoreMesh(core_axis_name="c", subcore_axis_name="s",
                                  num_cores=2, num_subcores=16)

    @functools.partial(
        pl.kernel,
        out_type=jax.ShapeDtypeStruct((NPAD * D,), jnp.float32),
        mesh=mesh,
        scratch_types=[
            pltpu.VMEM((N,), jnp.float32),
            pltpu.VMEM((96,), jnp.float32),
            pltpu.VMEM((96,), jnp.float32),
            pltpu.VMEM((144,), jnp.int32),
            pltpu.VMEM((CE,), jnp.int32),
            pltpu.VMEM((CE + KB,), jnp.int32),
            pltpu.VMEM((KB,), jnp.int32),
            pltpu.VMEM((KB,), jnp.int32),
            pltpu.VMEM((KB, D), jnp.float32),
            pltpu.VMEM((KB, D), jnp.float32),
            pltpu.VMEM(((NSR + 1) * D,), jnp.float32),
            pltpu.SemaphoreType.DMA,
            pltpu.SemaphoreType.DMA,
        ],
        compiler_params=pltpu.CompilerParams(needs_layout_passes=False,
                                             use_tc_tiling_on_sc=False),
    )
    def body(ft_h, el_h, er_h, dst_h, src_h, eb_h, out_h,
             el_v, er_v, em_v, eb_v, dst_v, src_v, idx0_v, idx1_v,
             row0_v, row1_v, stg_v, sem0, sem1):
        wid = lax.axis_index("s") * 2 + lax.axis_index("c")
        pltpu.sync_copy(el_h, el_v)
        pltpu.sync_copy(eb_h, eb_v)
        it = _I16()
        for q in range(KB // 16):
            src_v[pl.ds(CE + q * 16, 16)] = jnp.zeros((16,), jnp.int32)

        def range_body(r_i, _):
            r_glob = wid * RPW + r_i
            lo = r_glob * NSR
            es = _sget_i32(eb_v, r_glob)
            ee = _sget_i32(eb_v, r_glob + 1)
            a0 = (es // 8) * 8
            nedge = ee - es
            nck = (es - a0 + nedge + KB - 1) // KB
            nst = (nck + (CE // KB) - 1) // (CE // KB)

            pltpu.sync_copy(er_h.at[pl.ds(lo, 96)], er_v)

            def init_em(i, _c):
                em_v[pl.ds(i * 16, 16)] = jnp.full((16,), -3.4e38, jnp.float32)
                return _c
            lax.fori_loop(0, 6, init_em, 0)

            def init_stg(i, _c):
                stg_v[pl.ds(i * 16, 16)] = jnp.zeros((16,), jnp.float32)
                return _c
            lax.fori_loop(0, (NSR + 1) * D // 16, init_stg, 0)

            def load_keys(off):
                kraw = dst_v[pl.ds(off, 16)]
                sv = src_v[pl.ds(off, 16)]
                return kraw, sv

            def edge_logits(sv, kl):
                elg = plsc.load_gather(el_v, [sv])
                erg = plsc.load_gather(er_v, [kl])
                e = elg + erg
                return jnp.where(e > 0, e, 0.2 * e)

            def s1_stage(s, _c):
                pltpu.sync_copy(dst_h.at[pl.ds(a0 + s * CE, CE)], dst_v)
                pltpu.sync_copy(src_h.at[pl.ds(a0 + s * CE, CE)],
                                src_v.at[pl.ds(0, CE)])

                def s1_vec(v, _c2):
                    off = v * 16
                    ge = a0 + s * CE + off + it
                    valid = (ge >= es) & (ge < ee)
                    kraw, sv = load_keys(off)
                    kl = jnp.where(valid, kraw - lo, DUMMY)
                    e = edge_logits(sv, kl)
                    mx = _seg_scan(kl, e, jnp.maximum)
                    knext = _dg(kl, jnp.minimum(it + 1, 15))
                    endm = (kl != knext) | (it == 15)
                    cur = plsc.load_gather(em_v, [kl])
                    plsc.store_scatter(em_v, [kl], jnp.maximum(mx, cur),
                                       mask=endm)
                    return _c2

                nv = jnp.minimum((nck - s * (CE // KB)) * (KB // 16), CE // 16)
                lax.fori_loop(0, nv, s1_vec, 0)
                return _c
            lax.fori_loop(0, nst, s1_stage, 0)

            def s2_stage(s, carry):
                pltpu.sync_copy(dst_h.at[pl.ds(a0 + s * CE, CE)],
                                dst_v.at[pl.ds(0, CE)])
                pltpu.sync_copy(src_h.at[pl.ds(a0 + s * CE, CE)],
                                src_v.at[pl.ds(0, CE)])
                ncb = jnp.minimum(nck - s * CPS, CPS)
                nc2 = 2 * ((ncb + 1) // 2)

                def build_and_start(ci, idx_v, row_v, sem):
                    off0 = ci * KB
                    for q in range(KB // 16):
                        idx_v[pl.ds(q * 16, 16)] = (
                            src_v[pl.ds(off0 + q * 16, 16)])
                    pltpu.async_copy(ft_h.at[idx_v], row_v, sem)

                def s2_chunk(ci, row_v, carry2):
                    off0 = ci * KB

                    def s2_vec(b, carry3):
                        off = off0 + b * 16
                        ge = a0 + s * CE + off + it
                        valid = (ge >= es) & (ge < ee)
                        kraw, sv = load_keys(off)
                        kl = jnp.where(valid, kraw - lo, DUMMY)
                        e = edge_logits(sv, kl)
                        emg = plsc.load_gather(em_v, [kl])
                        un = jnp.exp(e - emg)
                        un = jnp.where(valid, un, 0.0)

                        acc, denc, prevk = carry3
                        shifted = _shift_up(kl, 1, prevk)
                        isnew = (kl != shifted).astype(jnp.int32)
                        dsum = _seg_scan(kl, un, jnp.add)
                        dsum = dsum + jnp.where(kl == prevk, denc, 0.0)
                        knext = _dg(kl, jnp.minimum(it + 1, 15))
                        endm = ((kl != knext) | (it == 15)).astype(jnp.int32)

                        def edge_body(j, acc_c):
                            accj = acc_c
                            w = _splat(un, j)
                            nsp = _splat(isnew, j) > 0
                            jj = b * 16 + j
                            new_acc = []
                            for k in range(D // 16):
                                rk = row_v[jj, pl.ds(k * 16, 16)]
                                a = jnp.where(nsp, w * rk, accj[k] + w * rk)
                                new_acc.append(a)
                            endrot = _dg(endm, (it + j) & 15)

                            @pl.when(endrot[0] > 0)
                            def _():
                                rsp = _splat(kl, j)
                                dsp = _splat(dsum, j)
                                rden = 1.0 / jnp.maximum(dsp, 1e-16)
                                for k in range(D // 16):
                                    plsc.store_scatter(
                                        stg_v, [rsp * D + k * 16 + it],
                                        new_acc[k] * rden)

                            return tuple(new_acc)

                        acc = lax.fori_loop(0, 16, edge_body, acc)
                        prevk = _splat(kl, 15)
                        denc = _splat(dsum, 15)
                        return (acc, denc, prevk)

                    return lax.fori_loop(0, 3, s2_vec, carry2)

                def wait_rows(idx_v, row_v, sem):
                    pltpu.make_async_copy(ft_h.at[idx_v], row_v, sem).wait()

                def pair_body(p, carry2):
                    c0 = 2 * p
                    build_and_start(c0 + 1, idx1_v, row1_v, sem1)
                    wait_rows(idx0_v, row0_v, sem0)
                    carry2 = s2_chunk(c0, row0_v, carry2)

                    @pl.when(c0 + 2 < nc2)
                    def _():
                        build_and_start(c0 + 2, idx0_v, row0_v, sem0)

                    wait_rows(idx1_v, row1_v, sem1)
                    return s2_chunk(c0 + 1, row1_v, carry2)

                @pl.when(nc2 > 0)
                def _():
                    build_and_start(0, idx0_v, row0_v, sem0)

                return lax.fori_loop(0, nc2 // 2, pair_body, carry)

            acc0 = tuple(jnp.zeros((16,), jnp.float32) for _ in range(D // 16))
            den0 = jnp.zeros((16,), jnp.float32)
            prev0 = jnp.full((16,), -1, jnp.int32)
            lax.fori_loop(0, nst, s2_stage, (acc0, den0, prev0))

            pltpu.sync_copy(stg_v.at[pl.ds(0, NSR * D)],
                            out_h.at[pl.ds(lo * D, NSR * D)])
            return _

        lax.fori_loop(0, RPW, range_body, 0)

    return body(ft, el, er_p, dst_p, src_p, ebnd)


def _tc_layer1(feat, W1, resW1, alv, arv):
    BR = 1000

    def body(x_ref, w_ref, rw_ref, al_ref, ar_ref,
             ft_ref, res_ref, el_ref, er_ref):
        ft = jnp.dot(x_ref[...], w_ref[...],
                     preferred_element_type=jnp.float32)
        ft_ref[...] = ft
        res_ref[...] = jnp.dot(x_ref[...], rw_ref[...],
                               preferred_element_type=jnp.float32)
        el_ref[...] = jnp.dot(ft, al_ref[...],
                              preferred_element_type=jnp.float32)
        er_ref[...] = jnp.dot(ft, ar_ref[...],
                              preferred_element_type=jnp.float32)

    IN = feat.shape[1]
    return pl.pallas_call(
        body,
        grid=(N // BR,),
        in_specs=[
            pl.BlockSpec((BR, IN), lambda i: (i, 0)),
            pl.BlockSpec((IN, D), lambda i: (0, 0)),
            pl.BlockSpec((IN, D), lambda i: (0, 0)),
            pl.BlockSpec((D, 1), lambda i: (0, 0)),
            pl.BlockSpec((D, 1), lambda i: (0, 0)),
        ],
        out_specs=[
            pl.BlockSpec((BR, D), lambda i: (i, 0)),
            pl.BlockSpec((BR, D), lambda i: (i, 0)),
            pl.BlockSpec((BR, 1), lambda i: (i, 0)),
            pl.BlockSpec((BR, 1), lambda i: (i, 0)),
        ],
        out_shape=[
            jax.ShapeDtypeStruct((N, D), jnp.float32),
            jax.ShapeDtypeStruct((N, D), jnp.float32),
            jax.ShapeDtypeStruct((N, 1), jnp.float32),
            jax.ShapeDtypeStruct((N, 1), jnp.float32),
        ],
    )(feat, W1, resW1, alv, arv)


def _tc_layer_mid(rst, res_prev, b_prev, W, alv, arv):
    BR = 1000

    def body(rst_ref, rp_ref, b_ref, w_ref, al_ref, ar_ref,
             x_ref, ft_ref, el_ref, er_ref):
        x = jnp.maximum(rst_ref[...] + rp_ref[...] + b_ref[...], 0.0)
        x_ref[...] = x
        ft = jnp.dot(x, w_ref[...], preferred_element_type=jnp.float32)
        ft_ref[...] = ft
        el_ref[...] = jnp.dot(ft, al_ref[...],
                              preferred_element_type=jnp.float32)
        er_ref[...] = jnp.dot(ft, ar_ref[...],
                              preferred_element_type=jnp.float32)

    return pl.pallas_call(
        body,
        grid=(N // BR,),
        in_specs=[
            pl.BlockSpec((BR, D), lambda i: (i, 0)),
            pl.BlockSpec((BR, D), lambda i: (i, 0)),
            pl.BlockSpec((1, D), lambda i: (0, 0)),
            pl.BlockSpec((D, D), lambda i: (0, 0)),
            pl.BlockSpec((D, 1), lambda i: (0, 0)),
            pl.BlockSpec((D, 1), lambda i: (0, 0)),
        ],
        out_specs=[
            pl.BlockSpec((BR, D), lambda i: (i, 0)),
            pl.BlockSpec((BR, D), lambda i: (i, 0)),
            pl.BlockSpec((BR, 1), lambda i: (i, 0)),
            pl.BlockSpec((BR, 1), lambda i: (i, 0)),
        ],
        out_shape=[
            jax.ShapeDtypeStruct((N, D), jnp.float32),
            jax.ShapeDtypeStruct((N, D), jnp.float32),
            jax.ShapeDtypeStruct((N, 1), jnp.float32),
            jax.ShapeDtypeStruct((N, 1), jnp.float32),
        ],
    )(rst, res_prev, b_prev, W, alv, arv)


def _tc_final(rst, res_prev, b_prev):
    BR = 1000

    def body(rst_ref, rp_ref, b_ref, o_ref):
        o_ref[...] = jnp.maximum(rst_ref[...] + rp_ref[...] + b_ref[...], 0.0)

    return pl.pallas_call(
        body,
        grid=(N // BR,),
        in_specs=[
            pl.BlockSpec((BR, D), lambda i: (i, 0)),
            pl.BlockSpec((BR, D), lambda i: (i, 0)),
            pl.BlockSpec((1, D), lambda i: (0, 0)),
        ],
        out_specs=pl.BlockSpec((BR, D), lambda i: (i, 0)),
        out_shape=jax.ShapeDtypeStruct((N, D), jnp.float32),
    )(rst, res_prev, b_prev)


def _prep_edges(edge_index):
    src = edge_index[0].astype(jnp.int32)
    dst = edge_index[1].astype(jnp.int32)
    dst_s, src_s = lax.sort((dst, src), num_keys=1)
    node_bounds = jnp.minimum(jnp.arange(R + 1, dtype=jnp.int32) * NSR, N)
    ebnd = jnp.searchsorted(dst_s, node_bounds, side="left").astype(jnp.int32)
    ebnd = jnp.pad(ebnd, (0, 144 - (R + 1)), constant_values=E)
    dst_p = jnp.pad(dst_s, (0, PADE - E), constant_values=N - 1)
    src_p = jnp.pad(src_s, (0, PADE - E), constant_values=0)
    return dst_p, src_p, ebnd


def _pad_er(er):
    return jnp.pad(er.reshape(N), (0, 10256 - N))


def kernel(feat, edge_index, W1, al1, ar1, b1, resW1, W2, al2, ar2, b2,
           W3, al3, ar3, b3, W4, al4, ar4, b4):
    dst_p, src_p, ebnd = _prep_edges(edge_index)

    def agg(ft, el, er):
        rst = _sc_aggregate(ft, el.reshape(N), _pad_er(er),
                            dst_p, src_p, ebnd)
        return rst.reshape(NPAD, D)[:N]

    ft1, res1, el1, er1 = _tc_layer1(
        feat, W1, resW1, al1.reshape(D, 1), ar1.reshape(D, 1))
    rst1 = agg(ft1, el1, er1)

    x1, ft2, el2, er2 = _tc_layer_mid(
        rst1, res1, b1.reshape(1, D), W2, al2.reshape(D, 1), ar2.reshape(D, 1))
    rst2 = agg(ft2, el2, er2)

    x2, ft3, el3, er3 = _tc_layer_mid(
        rst2, x1, b2.reshape(1, D), W3, al3.reshape(D, 1), ar3.reshape(D, 1))
    rst3 = agg(ft3, el3, er3)

    x3, ft4, el4, er4 = _tc_layer_mid(
        rst3, x2, b3.reshape(1, D), W4, al4.reshape(D, 1), ar4.reshape(D, 1))
    rst4 = agg(ft4, el4, er4)

    return _tc_final(rst4, x3, b4.reshape(1, D))
